# Initial kernel scaffold; baseline (speedup 1.0000x reference)
import jax
import jax.numpy as jnp
from jax.experimental import pallas as pl


def kernel(x_protocol, x_impression, x_treatment, edge_index_has, edge_index_suggests, edge_index_indicates, edge_index_issuggestedby, edge_weight_has, edge_weight_suggests, edge_weight_indicates, edge_weight_issuggestedby, params):
    raise NotImplementedError("write your pallas kernel here")


# speedup vs baseline: 1.1760x; 1.1760x over previous
import functools

import jax
import jax.numpy as jnp
from jax import lax
from jax.experimental import pallas as pl
from jax.experimental.pallas import tpu as pltpu
from jax.experimental.pallas import tpu_sc as plsc

N = 50000
D = 128
NC = 2
NS = 16
BE = 128
CH = 8448
NCHUNK = 6
NPAD = CH * NCHUNK
PASSES = NCHUNK // NC
CPT = CH // NS
ZR = 66
BR = 1000


def _pad_edges(ei, w):
    e = ei.shape[1]
    ept = ((e + NS * BE - 1) // (NS * BE)) * BE
    pad = ept * NS - e
    src = jnp.concatenate([ei[0], jnp.zeros((pad,), jnp.int32)])
    dst = jnp.concatenate([ei[1], jnp.zeros((pad,), jnp.int32)])
    wp = jnp.concatenate([w, jnp.zeros((pad,), jnp.float32)])
    return src, dst, wp, ept


@functools.cache
def _make_segsum(ept):
    nblk = ept // BE
    mesh = plsc.VectorSubcoreMesh(core_axis_name="c", subcore_axis_name="s")

    @functools.partial(
        pl.kernel,
        mesh=mesh,
        compiler_params=pltpu.CompilerParams(needs_layout_passes=False),
        out_type=jax.ShapeDtypeStruct((NPAD, D), jnp.float32),
        scratch_types=[
            pltpu.VMEM_SHARED((CH, D), jnp.float32),
            pltpu.VMEM((ZR, D), jnp.float32),
            pltpu.VMEM((BE,), jnp.int32),
            pltpu.VMEM((BE,), jnp.int32),
            pltpu.VMEM((BE,), jnp.float32),
            pltpu.VMEM((BE,), jnp.int32),
            pltpu.VMEM((BE,), jnp.int32),
            pltpu.VMEM((BE,), jnp.int32),
            pltpu.VMEM((BE,), jnp.int32),
            pltpu.VMEM((BE,), jnp.float32),
            pltpu.VMEM((BE,), jnp.float32),
            pltpu.VMEM((BE, D), jnp.float32),
            pltpu.SemaphoreType.DMA,
        ],
    )
    def seg(h_hbm, src_hbm, dst_hbm, w_hbm, out_hbm,
            acc, zbuf, sbuf, dbuf, wbuf,
            csA, csB, cdA, cdB, cwA, cwB, rows, sem):
        c = lax.axis_index("c")
        s = lax.axis_index("s")
        zero16f = jnp.zeros((16,), jnp.float32)
        zero16i = jnp.zeros((16,), jnp.int32)
        iota16 = lax.iota(jnp.int32, 16)

        def zb(i, carry):
            for u in range(8):
                zbuf[i, pl.ds(u * 16, 16)] = zero16f
            return carry

        lax.fori_loop(0, ZR, zb, 0)
        for g in range(8):
            sl = pl.ds(g * 16, 16)
            csA[sl] = zero16i
            csB[sl] = zero16i
            cdA[sl] = zero16i
            cdB[sl] = zero16i
            cwA[sl] = zero16f
            cwB[sl] = zero16f

        def fire_batch():
            pltpu.async_copy(h_hbm.at[csA], rows, sem).wait()

            def scale(j, carry2):
                wspl = plsc.load_gather(cwA, [jnp.full((16,), j, jnp.int32)])
                for u in range(8):
                    sl2 = pl.ds(u * 16, 16)
                    rows[j, sl2] = rows[j, sl2] * wspl
                return carry2

            lax.fori_loop(0, BE, scale, 0)
            pltpu.sync_copy(rows, acc.at[cdA], add=True)

        e0t = s * ept
        for p in range(PASSES):
            q = c * PASSES + p
            base = q * CH
            for k in range(CPT // ZR):
                pltpu.sync_copy(zbuf, acc.at[pl.ds(s * CPT + k * ZR, ZR)])
            plsc.subcore_barrier()

            def eblk(i, cnt):
                e0 = e0t + i * BE
                pltpu.sync_copy(src_hbm.at[pl.ds(e0, BE)], sbuf)
                pltpu.sync_copy(dst_hbm.at[pl.ds(e0, BE)], dbuf)
                pltpu.sync_copy(w_hbm.at[pl.ds(e0, BE)], wbuf)
                for g in range(8):
                    sl = pl.ds(g * 16, 16)
                    dv = dbuf[sl]
                    sv = sbuf[sl]
                    wv = wbuf[sl]
                    inm = (dv >= base) & (dv < base + CH)
                    ones = jnp.where(inm, 1, 0).astype(jnp.int32)
                    pos = cnt + plsc.cumsum(ones) - 1
                    posm = pos & (BE - 1)
                    in_a = inm & (pos < BE)
                    in_b = inm & (pos >= BE)
                    dloc = dv - base
                    plsc.store_scatter(csA, [posm], sv, mask=in_a)
                    plsc.store_scatter(csB, [posm], sv, mask=in_b)
                    plsc.store_scatter(cdA, [posm], dloc, mask=in_a)
                    plsc.store_scatter(cdB, [posm], dloc, mask=in_b)
                    plsc.store_scatter(cwA, [posm], wv, mask=in_a)
                    plsc.store_scatter(cwB, [posm], wv, mask=in_b)
                    cnt = cnt + plsc.all_reduce_population_count(inm)

                def fire(cv):
                    fire_batch()
                    for g2 in range(8):
                        sl2 = pl.ds(g2 * 16, 16)
                        csA[sl2] = csB[sl2]
                        cdA[sl2] = cdB[sl2]
                        cwA[sl2] = cwB[sl2]
                    return cv - BE

                cnt = lax.cond(jnp.max(cnt) >= BE, fire, lambda cv: cv, cnt)
                return cnt

            cnt = lax.fori_loop(0, nblk, eblk, zero16i)
            for g in range(8):
                sl = pl.ds(g * 16, 16)
                lane = iota16 + g * 16
                cwA[sl] = jnp.where(lane < cnt, cwA[sl], 0.0)
            fire_batch()
            plsc.subcore_barrier()
            pltpu.sync_copy(
                acc.at[pl.ds(s * CPT, CPT)],
                out_hbm.at[pl.ds(base + s * CPT, CPT)])
            plsc.subcore_barrier()

    return seg


def _mm(xs, ws, b, act):
    nin = len(xs)

    def body(*refs):
        in_refs = refs[:nin]
        w_refs = refs[nin:2 * nin]
        b_ref = refs[2 * nin]
        o_ref = refs[2 * nin + 1]
        acc = jnp.zeros((BR, D), jnp.float32)
        for xr, wr in zip(in_refs, w_refs):
            acc = acc + jnp.dot(xr[...], wr[...],
                                preferred_element_type=jnp.float32)
        acc = acc + b_ref[...]
        if act == "relu":
            acc = jnp.maximum(acc, 0.0)
        elif act == "lrelu":
            acc = jnp.where(acc > 0, acc, acc * 0.01)
        o_ref[...] = acc

    in_specs = (
        [pl.BlockSpec((BR, D), lambda i: (i, 0)) for _ in xs]
        + [pl.BlockSpec((D, D), lambda i: (0, 0)) for _ in ws]
        + [pl.BlockSpec((1, D), lambda i: (0, 0))]
    )
    f = pl.pallas_call(
        body,
        grid=(N // BR,),
        in_specs=in_specs,
        out_specs=pl.BlockSpec((BR, D), lambda i: (i, 0)),
        out_shape=jax.ShapeDtypeStruct((N, D), jnp.float32),
    )
    return f(*xs, *ws, b.reshape(1, D))


def kernel(x_protocol, x_impression, x_treatment, edge_index_has,
           edge_index_suggests, edge_index_indicates, edge_index_issuggestedby,
           edge_weight_has, edge_weight_suggests, edge_weight_indicates,
           edge_weight_issuggestedby, params):
    lin = params["lin"]
    h = {
        "protocol": _mm([x_protocol], [lin["protocol"]["W"]],
                        lin["protocol"]["b"], "relu"),
        "impression": _mm([x_impression], [lin["impression"]["W"]],
                          lin["impression"]["b"], "relu"),
        "treatment": _mm([x_treatment], [lin["treatment"]["W"]],
                         lin["treatment"]["b"], "relu"),
    }
    edges = {
        "has": _pad_edges(edge_index_has, edge_weight_has),
        "suggests": _pad_edges(edge_index_suggests, edge_weight_suggests),
        "indicates": _pad_edges(edge_index_indicates, edge_weight_indicates),
        "issuggestedby": _pad_edges(edge_index_issuggestedby,
                                    edge_weight_issuggestedby),
    }
    seg = _make_segsum(edges["has"][3])
    src_of = {"has": "protocol", "suggests": "protocol",
              "indicates": "impression", "issuggestedby": "treatment"}
    for layer in params["convs"]:
        agg = {et: seg(h[src_of[et]], *edges[et][:3]) for et in edges}
        new_i = _mm([agg["has"], h["impression"]],
                    [layer["has"]["W_rel"], layer["has"]["W_root"]],
                    layer["has"]["b_rel"], "lrelu")
        new_t = _mm([agg["suggests"], h["treatment"]],
                    [layer["suggests"]["W_rel"], layer["suggests"]["W_root"]],
                    layer["suggests"]["b_rel"], "lrelu")
        new_p = _mm(
            [agg["indicates"], agg["issuggestedby"], h["protocol"]],
            [layer["indicates"]["W_rel"], layer["issuggestedby"]["W_rel"],
             layer["indicates"]["W_root"] + layer["issuggestedby"]["W_root"]],
            layer["indicates"]["b_rel"] + layer["issuggestedby"]["b_rel"],
            "lrelu")
        h = {"protocol": new_p, "impression": new_i, "treatment": new_t}
    return _mm([h["protocol"]], [params["out"]["W"]], params["out"]["b"], None)
